# baseline (speedup 1.0000x reference)
import functools

import jax
import jax.numpy as jnp
from jax import lax
from jax.experimental import pallas as pl
from jax.experimental.pallas import tpu as pltpu
from jax.experimental.pallas import tpu_sc as plsc

EMBED = 2048
LANES = 16
NC, NS = 2, 16
NW = NC * NS
CHUNK = 8
UNROLL = 8


@functools.cache
def _build(B, S):
    TOT = B * S
    PER_W = TOT // NW
    NCHUNK = PER_W // CHUNK
    assert PER_W % CHUNK == 0 and S % PER_W == 0

    mesh = plsc.VectorSubcoreMesh(core_axis_name="c", subcore_axis_name="s")

    @functools.partial(
        pl.kernel,
        mesh=mesh,
        out_type=jax.ShapeDtypeStruct((TOT, EMBED), jnp.float32),
        scratch_types=[
            pltpu.VMEM((PER_W,), jnp.int32),
            pltpu.VMEM((CHUNK, EMBED), jnp.float32),
            pltpu.VMEM((CHUNK, EMBED), jnp.float32),
            pltpu.VMEM((CHUNK, EMBED), jnp.float32),
            pltpu.VMEM((CHUNK, EMBED), jnp.float32),
            pltpu.SemaphoreType.DMA,
            pltpu.SemaphoreType.DMA,
            pltpu.SemaphoreType.DMA,
            pltpu.SemaphoreType.DMA,
        ],
    )
    def emb_kernel(ids_hbm, table_hbm, pos_hbm, out_hbm,
                   idx_v, rows0, pos0, rows1, pos1,
                   semi0, semi1, semo0, semo1):
        wid = lax.axis_index("s") * NC + lax.axis_index("c")
        base = wid * PER_W
        s_base = lax.rem(base, S)
        pltpu.sync_copy(ids_hbm.at[pl.ds(base, PER_W)], idx_v)

        bufs = ((rows0, pos0, semi0, semo0), (rows1, pos1, semi1, semo1))

        def issue_in(i, rows_v, pos_v, sem):
            off = i * CHUNK
            g = pltpu.async_copy(
                table_hbm.at[idx_v.at[pl.ds(off, CHUNK)]], rows_v, sem)
            p = pltpu.async_copy(
                pos_hbm.at[pl.ds(s_base + off, CHUNK)], pos_v, sem)
            return g, p

        inflight = [None, None]
        stores = [None, None]
        inflight[0] = issue_in(0, *bufs[0][:3])
        for i in range(NCHUNK):
            b = i % 2
            nb = (i + 1) % 2
            if i + 1 < NCHUNK:
                if stores[nb] is not None:
                    stores[nb].wait()
                    stores[nb] = None
                inflight[nb] = issue_in(i + 1, *bufs[nb][:3])
            g, p = inflight[b]
            g.wait()
            p.wait()
            rows_v, pos_v, _, semo = bufs[b]

            @pl.loop(0, CHUNK)
            def _row(r):
                @pl.loop(0, EMBED, step=UNROLL * LANES)
                def _col(j):
                    slcs = [(pl.ds(r, 1), pl.ds(j + u * LANES, LANES))
                            for u in range(UNROLL)]
                    vals = [pos_v.at[slc][...] for slc in slcs]
                    for slc, v in zip(slcs, vals):
                        plsc.addupdate(rows_v.at[slc], v)

            stores[b] = pltpu.async_copy(
                rows_v, out_hbm.at[pl.ds(base + i * CHUNK, CHUNK)], semo)
        for st in stores:
            if st is not None:
                st.wait()

    return emb_kernel


@jax.jit
def kernel(token_ids, token_table, pos_table):
    B, S = token_ids.shape
    ids_flat = token_ids.reshape(B * S).astype(jnp.int32)
    out = _build(B, S)(ids_flat, token_table, pos_table[:S])
    return out.reshape(B, S, EMBED)


# speedup vs baseline: 1.0445x; 1.0445x over previous
import functools

import jax
import jax.numpy as jnp
from jax import lax
from jax.experimental import pallas as pl
from jax.experimental.pallas import tpu as pltpu
from jax.experimental.pallas import tpu_sc as plsc

EMBED = 2048
LANES = 16
NC, NS = 2, 16
NW = NC * NS
CHUNK = 8
UNROLL = 8
NBUF = 3


@functools.cache
def _build(B, S):
    TOT = B * S
    PER_W = TOT // NW
    NCHUNK = PER_W // CHUNK
    assert PER_W % CHUNK == 0 and S % PER_W == 0

    mesh = plsc.VectorSubcoreMesh(core_axis_name="c", subcore_axis_name="s")

    scratch = [pltpu.VMEM((PER_W,), jnp.int32)]
    for _ in range(NBUF):
        scratch.append(pltpu.VMEM((CHUNK, EMBED), jnp.float32))
        scratch.append(pltpu.VMEM((CHUNK, EMBED), jnp.float32))
        scratch.append(pltpu.SemaphoreType.DMA)
        scratch.append(pltpu.SemaphoreType.DMA)

    @functools.partial(
        pl.kernel,
        mesh=mesh,
        out_type=jax.ShapeDtypeStruct((TOT, EMBED), jnp.float32),
        scratch_types=scratch,
    )
    def emb_kernel(ids_hbm, table_hbm, pos_hbm, out_hbm, idx_v, *bufflat):
        wid = lax.axis_index("s") * NC + lax.axis_index("c")
        base = wid * PER_W
        s_base = lax.rem(base, S)
        pltpu.sync_copy(ids_hbm.at[pl.ds(base, PER_W)], idx_v)

        bufs = [tuple(bufflat[4 * k: 4 * k + 4]) for k in range(NBUF)]

        def issue_in(i, rows_v, pos_v, semi, semo):
            off = i * CHUNK
            g = pltpu.async_copy(
                table_hbm.at[idx_v.at[pl.ds(off, CHUNK)]], rows_v, semi)
            p = pltpu.async_copy(
                pos_hbm.at[pl.ds(s_base + off, CHUNK)], pos_v, semi)
            return g, p

        inflight = [None] * NBUF
        stores = [None] * NBUF
        for k in range(min(NBUF - 1, NCHUNK)):
            inflight[k] = issue_in(k, *bufs[k])
        for i in range(NCHUNK):
            b = i % NBUF
            if i + NBUF - 1 < NCHUNK:
                nb = (i + NBUF - 1) % NBUF
                if stores[nb] is not None:
                    stores[nb].wait()
                    stores[nb] = None
                inflight[nb] = issue_in(i + NBUF - 1, *bufs[nb])
            g, p = inflight[b]
            g.wait()
            p.wait()
            rows_v, pos_v, _, semo = bufs[b]

            @pl.loop(0, CHUNK)
            def _row(r):
                @pl.loop(0, EMBED, step=UNROLL * LANES)
                def _col(j):
                    slcs = [(pl.ds(r, 1), pl.ds(j + u * LANES, LANES))
                            for u in range(UNROLL)]
                    vals = [pos_v.at[slc][...] for slc in slcs]
                    for slc, v in zip(slcs, vals):
                        plsc.addupdate(rows_v.at[slc], v)

            if stores[b] is not None:
                stores[b].wait()
            stores[b] = pltpu.async_copy(
                rows_v, out_hbm.at[pl.ds(base + i * CHUNK, CHUNK)], semo)
        for st in stores:
            if st is not None:
                st.wait()

    return emb_kernel


@jax.jit
def kernel(token_ids, token_table, pos_table):
    B, S = token_ids.shape
    ids_flat = token_ids.reshape(B * S).astype(jnp.int32)
    out = _build(B, S)(ids_flat, token_table, pos_table[:S])
    return out.reshape(B, S, EMBED)


# speedup vs baseline: 1.0448x; 1.0003x over previous
import functools

import jax
import jax.numpy as jnp
from jax import lax
from jax.experimental import pallas as pl
from jax.experimental.pallas import tpu as pltpu
from jax.experimental.pallas import tpu_sc as plsc

EMBED = 2048
LANES = 16
NC, NS = 2, 16
NW = NC * NS
CHUNK = 8
UNROLL = 8
NBUF = 3


@functools.cache
def _build(B, S):
    TOT = B * S
    PER_W = TOT // NW
    NCHUNK = PER_W // CHUNK
    assert PER_W % CHUNK == 0 and S % PER_W == 0

    mesh = plsc.VectorSubcoreMesh(core_axis_name="c", subcore_axis_name="s")

    scratch = [pltpu.VMEM((NCHUNK, CHUNK), jnp.int32)]
    for _ in range(NBUF):
        scratch.append(pltpu.VMEM((CHUNK, EMBED), jnp.float32))
        scratch.append(pltpu.VMEM((CHUNK, EMBED), jnp.float32))
        scratch.append(pltpu.SemaphoreType.DMA)
        scratch.append(pltpu.SemaphoreType.DMA)

    @functools.partial(
        pl.kernel,
        mesh=mesh,
        out_type=jax.ShapeDtypeStruct((TOT, EMBED), jnp.float32),
        scratch_types=scratch,
    )
    def emb_kernel(ids_hbm, table_hbm, pos_hbm, out_hbm, idx_v, *bufflat):
        wid = lax.axis_index("s") * NC + lax.axis_index("c")
        base = wid * PER_W
        s_base = lax.rem(base, S)
        pltpu.sync_copy(ids_hbm.at[wid], idx_v)

        bufs = [tuple(bufflat[4 * k: 4 * k + 4]) for k in range(NBUF)]

        def issue_in(i, rows_v, pos_v, semi, semo):
            off = i * CHUNK
            g = pltpu.async_copy(
                table_hbm.at[idx_v.at[i]], rows_v, semi)
            p = pltpu.async_copy(
                pos_hbm.at[pl.ds(s_base + off, CHUNK)], pos_v, semi)
            return g, p

        inflight = [None] * NBUF
        stores = [None] * NBUF
        for k in range(min(NBUF - 1, NCHUNK)):
            inflight[k] = issue_in(k, *bufs[k])
        for i in range(NCHUNK):
            b = i % NBUF
            if i + NBUF - 1 < NCHUNK:
                nb = (i + NBUF - 1) % NBUF
                if stores[nb] is not None:
                    stores[nb].wait()
                    stores[nb] = None
                inflight[nb] = issue_in(i + NBUF - 1, *bufs[nb])
            g, p = inflight[b]
            g.wait()
            p.wait()
            rows_v, pos_v, _, semo = bufs[b]

            @pl.loop(0, CHUNK)
            def _row(r):
                @pl.loop(0, EMBED, step=UNROLL * LANES)
                def _col(j):
                    slcs = [(pl.ds(r, 1), pl.ds(j + u * LANES, LANES))
                            for u in range(UNROLL)]
                    vals = [pos_v.at[slc][...] for slc in slcs]
                    for slc, v in zip(slcs, vals):
                        plsc.addupdate(rows_v.at[slc], v)

            if stores[b] is not None:
                stores[b].wait()
            stores[b] = pltpu.async_copy(
                rows_v, out_hbm.at[pl.ds(base + i * CHUNK, CHUNK)], semo)
        for st in stores:
            if st is not None:
                st.wait()

    return emb_kernel


@jax.jit
def kernel(token_ids, token_table, pos_table):
    B, S = token_ids.shape
    ids3 = token_ids.reshape(NW, (B * S) // (NW * CHUNK), CHUNK).astype(jnp.int32)
    out = _build(B, S)(ids3, token_table, pos_table[:S])
    return out.reshape(B, S, EMBED)
